# Initial kernel scaffold; baseline (speedup 1.0000x reference)
import jax
import jax.numpy as jnp
from jax.experimental import pallas as pl


def kernel(x, embd):
    raise NotImplementedError("write your pallas kernel here")


# speedup vs baseline: 2.9598x; 2.9598x over previous
import functools

import jax
import jax.numpy as jnp
from jax import lax
from jax.experimental import pallas as pl
from jax.experimental.pallas import tpu as pltpu
from jax.experimental.pallas import tpu_sc as plsc

D = 128
B = 4096 * 50
NC = 2
NS = 16
NW = NC * NS
BPW = B // NW
CH = 128
NCH = BPW // CH

_mesh = plsc.VectorSubcoreMesh(core_axis_name="c", subcore_axis_name="s")


@functools.partial(
    pl.kernel,
    mesh=_mesh,
    out_type=jax.ShapeDtypeStruct((B, D), jnp.float32),
    scratch_types=[
        pltpu.VMEM((BPW,), jnp.int32),
        pltpu.VMEM((CH, D), jnp.float32),
        pltpu.SemaphoreType.DMA,
    ],
)
def _gather_kernel(x_hbm, embd_hbm, out_hbm, idx_v, buf, gsem):
    wid = lax.axis_index("s") * NC + lax.axis_index("c")
    base = pl.multiple_of(wid * BPW, CH)
    pltpu.sync_copy(x_hbm.at[pl.ds(base, BPW)], idx_v)

    def body(j, carry):
        off = pl.multiple_of(j * CH, CH)
        pltpu.async_copy(
            embd_hbm.at[idx_v.at[pl.ds(off, CH)]], buf, gsem
        ).wait()
        pltpu.sync_copy(buf, out_hbm.at[pl.ds(base + off, CH)])
        return carry

    lax.fori_loop(0, NCH, body, 0)


def kernel(x, embd):
    flat = x.reshape(-1).astype(jnp.int32)
    out = _gather_kernel(flat, embd)
    return out.reshape(x.shape + (D,))


# speedup vs baseline: 3.2905x; 1.1117x over previous
import functools

import jax
import jax.numpy as jnp
from jax import lax
from jax.experimental import pallas as pl
from jax.experimental.pallas import tpu as pltpu
from jax.experimental.pallas import tpu_sc as plsc

D = 128
B = 4096 * 50
NC = 2
NS = 16
NW = NC * NS
BPW = B // NW
CH = 128
GCH = 2 * CH
NG = BPW // GCH

_mesh = plsc.VectorSubcoreMesh(core_axis_name="c", subcore_axis_name="s")


@functools.partial(
    pl.kernel,
    mesh=_mesh,
    out_type=jax.ShapeDtypeStruct((B, D), jnp.float32),
    scratch_types=[
        pltpu.VMEM((BPW,), jnp.int32),
        pltpu.VMEM((GCH, D), jnp.float32),
        pltpu.VMEM((GCH, D), jnp.float32),
        pltpu.SemaphoreType.DMA,
        pltpu.SemaphoreType.DMA,
    ],
)
def _gather_kernel(x_hbm, embd_hbm, out_hbm, idx_v, buf_a, buf_b, gsem, ssem):
    wid = lax.axis_index("s") * NC + lax.axis_index("c")
    base = pl.multiple_of(wid * BPW, 8)
    pltpu.sync_copy(x_hbm.at[pl.ds(base, BPW)], idx_v)

    def gather_desc(g, buf, b):
        off = pl.multiple_of(g * GCH + b * CH, 8)
        return pltpu.make_async_copy(
            embd_hbm.at[idx_v.at[pl.ds(off, CH)]],
            buf.at[pl.ds(b * CH, CH)],
            gsem,
        )

    def store_desc(g, buf):
        off = pl.multiple_of(base + g * GCH, 8)
        return pltpu.make_async_copy(buf, out_hbm.at[pl.ds(off, GCH)], ssem)

    def fire_gathers(g, buf):
        for b in range(GCH // CH):
            gather_desc(g, buf, b).start()

    def wait_gathers(g, buf):
        for b in range(GCH // CH):
            gather_desc(g, buf, b).wait()

    fire_gathers(0, buf_a)
    wait_gathers(0, buf_a)
    fire_gathers(1, buf_b)
    store_desc(0, buf_a).start()

    def step(g, cur, other):
        wait_gathers(g, cur)
        store_desc(g - 1, other).wait()
        fire_gathers(g + 1, other)
        store_desc(g, cur).start()

    def body(m, carry):
        step(2 * m + 1, buf_b, buf_a)
        step(2 * m + 2, buf_a, buf_b)
        return carry

    lax.fori_loop(0, (NG - 3) // 2, body, 0)

    step(NG - 2, buf_b, buf_a)
    wait_gathers(NG - 1, buf_a)
    store_desc(NG - 2, buf_b).wait()
    store_desc(NG - 1, buf_a).start()
    store_desc(NG - 1, buf_a).wait()


def kernel(x, embd):
    flat = x.reshape(-1).astype(jnp.int32)
    out = _gather_kernel(flat, embd)
    return out.reshape(x.shape + (D,))


# speedup vs baseline: 5.8297x; 1.7717x over previous
import functools

import jax
import jax.numpy as jnp
from jax import lax
from jax.experimental import pallas as pl
from jax.experimental.pallas import tpu as pltpu
from jax.experimental.pallas import tpu_sc as plsc

D = 128
NP = 4096
PW = 50
NC = 2
NS = 16
NW = NC * NS
PPW = NP // NW
BPW = PPW * PW
GP = 8
GCH = GP * PW
SCH = 80
NSTR = GCH // SCH
NG = PPW // GP

_mesh = plsc.VectorSubcoreMesh(core_axis_name="c", subcore_axis_name="s")


@functools.partial(
    pl.kernel,
    mesh=_mesh,
    out_type=jax.ShapeDtypeStruct((NP, PW, D), jnp.float32),
    scratch_types=[
        pltpu.VMEM((BPW,), jnp.int32),
        pltpu.VMEM((GCH, D), jnp.float32),
        pltpu.VMEM((GCH, D), jnp.float32),
        pltpu.SemaphoreType.DMA,
        pltpu.SemaphoreType.DMA,
    ],
)
def _gather_kernel(x_hbm, embd_hbm, out_hbm, idx_v, buf_a, buf_b, gsem, ssem):
    wid = lax.axis_index("s") * NC + lax.axis_index("c")
    base = pl.multiple_of(wid * BPW, 8)
    pbase = wid * PPW
    pltpu.sync_copy(x_hbm.at[pl.ds(base, BPW)], idx_v)

    def gather_desc(g, buf, s):
        off = pl.multiple_of(g * GCH + s * SCH, 8)
        return pltpu.make_async_copy(
            embd_hbm.at[idx_v.at[pl.ds(off, SCH)]],
            buf.at[pl.ds(s * SCH, SCH)],
            gsem,
        )

    def store_desc(g, buf, p):
        return pltpu.make_async_copy(
            buf.at[pl.ds(p * PW, PW)],
            out_hbm.at[pbase + g * GP + p],
            ssem,
        )

    def fire_gathers(g, buf):
        for s in range(NSTR):
            gather_desc(g, buf, s).start()

    def wait_gathers(g, buf):
        for s in range(NSTR):
            gather_desc(g, buf, s).wait()

    def fire_stores(g, buf):
        for p in range(GP):
            store_desc(g, buf, p).start()

    def wait_stores(g, buf):
        for p in range(GP):
            store_desc(g, buf, p).wait()

    fire_gathers(0, buf_a)
    wait_gathers(0, buf_a)
    fire_gathers(1, buf_b)
    fire_stores(0, buf_a)

    def step(g, cur, other):
        wait_gathers(g, cur)
        wait_stores(g - 1, other)
        fire_gathers(g + 1, other)
        fire_stores(g, cur)

    def body(m, carry):
        step(2 * m + 1, buf_b, buf_a)
        step(2 * m + 2, buf_a, buf_b)
        return carry

    lax.fori_loop(0, (NG - 2) // 2, body, 0)

    wait_gathers(NG - 1, buf_b)
    wait_stores(NG - 2, buf_a)
    fire_stores(NG - 1, buf_b)
    wait_stores(NG - 1, buf_b)


def kernel(x, embd):
    flat = x.reshape(-1).astype(jnp.int32)
    return _gather_kernel(flat, embd)


# speedup vs baseline: 5.8351x; 1.0009x over previous
import functools

import jax
import jax.numpy as jnp
from jax import lax
from jax.experimental import pallas as pl
from jax.experimental.pallas import tpu as pltpu
from jax.experimental.pallas import tpu_sc as plsc

D = 128
NP = 4096
PW = 50
NC = 2
NS = 16
NW = NC * NS
PPW = NP // NW
BPW = PPW * PW
GP = 8
GCH = GP * PW
SCH = 80
NSTR = GCH // SCH
NG = PPW // GP

_mesh = plsc.VectorSubcoreMesh(core_axis_name="c", subcore_axis_name="s")


@functools.partial(
    pl.kernel,
    mesh=_mesh,
    compiler_params=pltpu.CompilerParams(use_tc_tiling_on_sc=True),
    out_type=jax.ShapeDtypeStruct((NP, PW, D), jnp.float32),
    scratch_types=[
        pltpu.VMEM((BPW,), jnp.int32),
        pltpu.VMEM((GCH, D), jnp.float32),
        pltpu.VMEM((GCH, D), jnp.float32),
        pltpu.SemaphoreType.DMA,
        pltpu.SemaphoreType.DMA,
    ],
)
def _gather_kernel(x_hbm, embd_hbm, out_hbm, idx_v, buf_a, buf_b, gsem, ssem):
    wid = lax.axis_index("s") * NC + lax.axis_index("c")
    base = pl.multiple_of(wid * BPW, 8)
    pbase = wid * PPW
    pltpu.sync_copy(x_hbm.at[pl.ds(base, BPW)], idx_v)

    def gather_desc(g, buf, s):
        off = pl.multiple_of(g * GCH + s * SCH, 8)
        return pltpu.make_async_copy(
            embd_hbm.at[idx_v.at[pl.ds(off, SCH)]],
            buf.at[pl.ds(s * SCH, SCH)],
            gsem,
        )

    def store_desc(g, buf, p):
        return pltpu.make_async_copy(
            buf.at[pl.ds(p * PW, PW)],
            out_hbm.at[pbase + g * GP + p],
            ssem,
        )

    def fire_gathers(g, buf):
        for s in range(NSTR):
            gather_desc(g, buf, s).start()

    def wait_gathers(g, buf):
        for s in range(NSTR):
            gather_desc(g, buf, s).wait()

    def fire_stores(g, buf):
        for p in range(GP):
            store_desc(g, buf, p).start()

    def wait_stores(g, buf):
        for p in range(GP):
            store_desc(g, buf, p).wait()

    fire_gathers(0, buf_a)
    wait_gathers(0, buf_a)
    fire_gathers(1, buf_b)
    fire_stores(0, buf_a)

    def step(g, cur, other):
        wait_gathers(g, cur)
        wait_stores(g - 1, other)
        fire_gathers(g + 1, other)
        fire_stores(g, cur)

    def body(m, carry):
        step(2 * m + 1, buf_b, buf_a)
        step(2 * m + 2, buf_a, buf_b)
        return carry

    lax.fori_loop(0, (NG - 2) // 2, body, 0)

    wait_gathers(NG - 1, buf_b)
    wait_stores(NG - 2, buf_a)
    fire_stores(NG - 1, buf_b)
    wait_stores(NG - 1, buf_b)


def kernel(x, embd):
    flat = x.reshape(-1).astype(jnp.int32)
    return _gather_kernel(flat, embd)
